# baseline (speedup 1.0000x reference)
import jax
import jax.numpy as jnp
from jax import lax
from jax.experimental import pallas as pl
from jax.experimental.pallas import tpu as pltpu
from jax.experimental.pallas import tpu_sc as plsc

WH = 16
NTOK = 1025
N = NTOK * NTOK
NDIST = 3972
L = 16
NW = 32
NBLK = 128
BPW = NBLK // NW
BROW = 8 * NTOK
RPAD = 1032


def _body(table_hbm, idx_hbm, out_hbm, row_hbm, col_hbm,
          table_v, idx_v, vals_v, rowi_v, rowv_v, colv_v):
    wid = lax.axis_index("s") * 2 + lax.axis_index("c")
    viota = lax.iota(jnp.int32, L)
    pltpu.sync_copy(table_hbm, table_v)

    def block(b, carry):
        blk = wid * BPW + b
        pltpu.sync_copy(idx_hbm.at[pl.ds(blk * BROW, BROW)], idx_v)
        for h in range(WH):

            def group(g, carry2):
                for rr in range(8):
                    vidx = idx_v[pl.ds(rr * NTOK + g * L, L)]
                    vals_v[rr, pl.ds(g * L, L)] = plsc.load_gather(
                        table_v, [vidx * WH + h])
                return carry2

            lax.fori_loop(0, 64, group, 0)
            pltpu.sync_copy(vals_v,
                            out_hbm.at[h, pl.ds(blk * 8, 8), pl.ds(0, 1024)])
            vca = jnp.minimum(viota * NTOK + (NTOK - 1), BROW - 1)
            vcidx = plsc.load_gather(idx_v, [vca])
            cv = plsc.load_gather(table_v, [vcidx * WH + h])
            plsc.store_scatter(colv_v, [h * (8 * BPW) + b * 8 + viota], cv,
                               mask=viota < 8)
        return carry

    lax.fori_loop(0, BPW, block, 0)
    for h in range(WH):
        pltpu.sync_copy(colv_v.at[pl.ds(h * (8 * BPW), 8 * BPW)],
                        col_hbm.at[pl.ds(h * 1024 + wid * (8 * BPW), 8 * BPW)])

    @pl.when(wid < WH)
    def _tail_row():
        rowi_v[pl.ds(1024, L)] = jnp.zeros((L,), jnp.int32)
        pltpu.sync_copy(idx_hbm.at[pl.ds(1024 * NTOK, 1024)],
                        rowi_v.at[pl.ds(0, 1024)])
        pltpu.sync_copy(idx_hbm.at[pl.ds(N - 1, 1)], rowi_v.at[pl.ds(1024, 1)])

        def rgroup(g, carry):
            vri = rowi_v[pl.ds(g * L, L)]
            rowv_v[pl.ds(g * L, L)] = plsc.load_gather(table_v,
                                                       [vri * WH + wid])
            return carry

        lax.fori_loop(0, 65, rgroup, 0)
        pltpu.sync_copy(rowv_v.at[pl.ds(0, RPAD)],
                        row_hbm.at[pl.ds(wid * RPAD, RPAD)])


@jax.jit
def _launch(table, idx32):
    mesh = plsc.VectorSubcoreMesh(core_axis_name="c", subcore_axis_name="s")
    f = pl.kernel(
        _body,
        out_type=(
            jax.ShapeDtypeStruct((WH, NTOK, NTOK), jnp.float32),
            jax.ShapeDtypeStruct((WH * RPAD,), jnp.float32),
            jax.ShapeDtypeStruct((WH * 1024,), jnp.float32),
        ),
        mesh=mesh,
        compiler_params=pltpu.CompilerParams(needs_layout_passes=False),
        scratch_types=[
            pltpu.VMEM((NDIST * WH,), jnp.float32),
            pltpu.VMEM((BROW,), jnp.int32),
            pltpu.VMEM((8, 1024), jnp.float32),
            pltpu.VMEM((1040,), jnp.int32),
            pltpu.VMEM((1040,), jnp.float32),
            pltpu.VMEM((WH * 8 * BPW,), jnp.float32),
        ],
    )
    return f(table, idx32)


def kernel(relative_position_bias_table, relative_position_index):
    idx32 = relative_position_index.reshape(-1).astype(jnp.int32)
    out, aux_row, aux_col = _launch(relative_position_bias_table.reshape(-1),
                                    idx32)
    tail_row = aux_row.reshape(WH, RPAD)[:, :NTOK]
    tail_col = aux_col.reshape(WH, 1024)
    out = out.at[:, NTOK - 1, :].set(tail_row)
    out = out.at[:, :1024, NTOK - 1].set(tail_col)
    return out


# speedup vs baseline: 1.8903x; 1.8903x over previous
import jax
import jax.numpy as jnp
from jax import lax
from jax.experimental import pallas as pl
from jax.experimental.pallas import tpu as pltpu
from jax.experimental.pallas import tpu_sc as plsc

WH = 16
NTOK = 1025
N = NTOK * NTOK
NDIST = 3972
L = 16
NW = 32
NBLK = 128
BPW = NBLK // NW
BROW = 8 * NTOK
RPAD = 1032


def _body(table_hbm, idx_hbm, out_hbm, row_hbm, col_hbm,
          table_v, idx_v, vals_v, rowi_v, rowv_v, colv_v, sem):
    wid = lax.axis_index("s") * 2 + lax.axis_index("c")
    viota = lax.iota(jnp.int32, L)
    pltpu.sync_copy(table_hbm, table_v)

    def wait_one():
        pltpu.make_async_copy(
            vals_v.at[0], out_hbm.at[0, pl.ds(0, 8), pl.ds(0, 1024)],
            sem).wait()

    def block(b, carry):
        blk = wid * BPW + b
        pltpu.sync_copy(idx_hbm.at[pl.ds(blk * BROW, BROW)], idx_v)
        for h in range(WH):
            if h < 2:
                @pl.when(b > 0)
                def _():
                    wait_one()
            else:
                wait_one()
            p = h % 2

            @plsc.parallel_loop(0, 64, unroll=2)
            def group(g):
                for rr in range(8):
                    vidx = idx_v[pl.ds(rr * NTOK + g * L, L)]
                    vals_v[p, rr, pl.ds(g * L, L)] = plsc.load_gather(
                        table_v, [vidx * WH + h])

            pltpu.async_copy(
                vals_v.at[p],
                out_hbm.at[h, pl.ds(blk * 8, 8), pl.ds(0, 1024)], sem)
            vca = jnp.minimum(viota * NTOK + (NTOK - 1), BROW - 1)
            vcidx = plsc.load_gather(idx_v, [vca])
            cv = plsc.load_gather(table_v, [vcidx * WH + h])
            plsc.store_scatter(colv_v, [h * (8 * BPW) + b * 8 + viota], cv,
                               mask=viota < 8)
        return carry

    lax.fori_loop(0, BPW, block, 0)
    wait_one()
    wait_one()
    for h in range(WH):
        pltpu.sync_copy(colv_v.at[pl.ds(h * (8 * BPW), 8 * BPW)],
                        col_hbm.at[pl.ds(h * 1024 + wid * (8 * BPW), 8 * BPW)])

    @pl.when(wid < WH)
    def _tail_row():
        rowi_v[pl.ds(1024, L)] = jnp.zeros((L,), jnp.int32)
        pltpu.sync_copy(idx_hbm.at[pl.ds(1024 * NTOK, 1024)],
                        rowi_v.at[pl.ds(0, 1024)])
        pltpu.sync_copy(idx_hbm.at[pl.ds(N - 1, 1)], rowi_v.at[pl.ds(1024, 1)])

        @plsc.parallel_loop(0, 65, unroll=2)
        def rgroup(g):
            vri = rowi_v[pl.ds(g * L, L)]
            rowv_v[pl.ds(g * L, L)] = plsc.load_gather(table_v,
                                                       [vri * WH + wid])

        pltpu.sync_copy(rowv_v.at[pl.ds(0, RPAD)],
                        row_hbm.at[pl.ds(wid * RPAD, RPAD)])


@jax.jit
def _launch(table, idx32):
    mesh = plsc.VectorSubcoreMesh(core_axis_name="c", subcore_axis_name="s")
    f = pl.kernel(
        _body,
        out_type=(
            jax.ShapeDtypeStruct((WH, NTOK, NTOK), jnp.float32),
            jax.ShapeDtypeStruct((WH * RPAD,), jnp.float32),
            jax.ShapeDtypeStruct((WH * 1024,), jnp.float32),
        ),
        mesh=mesh,
        compiler_params=pltpu.CompilerParams(needs_layout_passes=False),
        scratch_types=[
            pltpu.VMEM((NDIST * WH,), jnp.float32),
            pltpu.VMEM((BROW,), jnp.int32),
            pltpu.VMEM((2, 8, 1024), jnp.float32),
            pltpu.VMEM((1040,), jnp.int32),
            pltpu.VMEM((1040,), jnp.float32),
            pltpu.VMEM((WH * 8 * BPW,), jnp.float32),
            pltpu.SemaphoreType.DMA,
        ],
    )
    return f(table, idx32)


def kernel(relative_position_bias_table, relative_position_index):
    idx32 = relative_position_index.reshape(-1).astype(jnp.int32)
    out, aux_row, aux_col = _launch(relative_position_bias_table.reshape(-1),
                                    idx32)
    tail_row = aux_row.reshape(WH, RPAD)[:, :NTOK]
    tail_col = aux_col.reshape(WH, 1024)
    out = out.at[:, NTOK - 1, :].set(tail_row)
    out = out.at[:, :1024, NTOK - 1].set(tail_col)
    return out
